# baseline (speedup 1.0000x reference)
import functools

import jax
import jax.numpy as jnp
from jax import lax
from jax.experimental import pallas as pl
from jax.experimental.pallas import tpu as pltpu
from jax.experimental.pallas import tpu_sc as plsc

N = 10000
E = 320000
CH = 128
NPAD = 10240
EP = 327680
XW = 4
NC = 2
NS = 16
NW = NC * NS
EPW = EP // NW
CHUNK = 128
NCHUNK = EPW // CHUNK
RPT = NPAD // NS
XF = NPAD * 3

BE = 2048
BN = 1024


def _silu(x):
    return x * jax.nn.sigmoid(x)


def _mm(a, b):
    return jnp.dot(a, b, preferred_element_type=jnp.float32)


_sc_cache = {}


def _build_sc_kernels():
    if _sc_cache:
        return _sc_cache['gather'], _sc_cache['scatter']

    mesh = plsc.VectorSubcoreMesh(core_axis_name="c", subcore_axis_name="s")

    @functools.partial(
        pl.kernel,
        mesh=mesh,
        out_type=(
            jax.ShapeDtypeStruct((EP, CH // 2), jnp.int32),
            jax.ShapeDtypeStruct((EP, CH // 2), jnp.int32),
            jax.ShapeDtypeStruct((EP * XW,), jnp.float32),
        ),
        scratch_types=(
            pltpu.VMEM((EPW,), jnp.int32),
            pltpu.VMEM((EPW,), jnp.int32),
            pltpu.VMEM((CHUNK, CH // 2), jnp.int32),
            pltpu.VMEM((CHUNK, CH // 2), jnp.int32),
            pltpu.VMEM((CHUNK * XW,), jnp.float32),
            pltpu.VMEM((CHUNK, CH // 2), jnp.int32),
            pltpu.VMEM((CHUNK, CH // 2), jnp.int32),
            pltpu.VMEM((CHUNK * XW,), jnp.float32),
            pltpu.VMEM((XF,), jnp.float32),
            pltpu.SemaphoreType.DMA,
            pltpu.SemaphoreType.DMA,
            pltpu.SemaphoreType.DMA,
        ),
        compiler_params=pltpu.CompilerParams(
            needs_layout_passes=False, use_tc_tiling_on_sc=False),
    )
    def _sc_gather(p_hbm, q_hbm, x_hbm, src_hbm, dst_hbm,
                   g_hbm, h_hbm, xd_hbm,
                   s_all, d_all, gb0, hb0, db0, gb1, hb1, db1, xtab,
                   semg, semw0, semw1):
        wid = lax.axis_index("s") * NC + lax.axis_index("c")
        base = wid * EPW
        iota16 = lax.iota(jnp.int32, 16)
        pltpu.sync_copy(x_hbm, xtab)
        pltpu.sync_copy(src_hbm.at[pl.ds(base, EPW)], s_all)
        pltpu.sync_copy(dst_hbm.at[pl.ds(base, EPW)], d_all)

        def issue(k, gb, hb):
            sl = pl.ds(k * CHUNK, CHUNK)
            c1 = pltpu.async_copy(p_hbm.at[s_all.at[sl]], gb, semg)
            c2 = pltpu.async_copy(q_hbm.at[d_all.at[sl]], hb, semg)
            return c1, c2

        def xcompute(k, db):
            for g in range(CHUNK // 16):
                o = k * CHUNK + g * 16
                rs = s_all[pl.ds(o, 16)] * 3
                rd = d_all[pl.ds(o, 16)] * 3
                dx = (plsc.load_gather(xtab, [rs]) -
                      plsc.load_gather(xtab, [rd]))
                dy = (plsc.load_gather(xtab, [rs + 1]) -
                      plsc.load_gather(xtab, [rd + 1]))
                dz_ = (plsc.load_gather(xtab, [rs + 2]) -
                       plsc.load_gather(xtab, [rd + 2]))
                radial = dx * dx + dy * dy + dz_ * dz_
                row = (iota16 + g * 16) * XW
                plsc.store_scatter(db, [row], dx)
                plsc.store_scatter(db, [row + 1], dy)
                plsc.store_scatter(db, [row + 2], dz_)
                plsc.store_scatter(db, [row + 3], radial)

        def write(k, gb, hb, db, semw):
            off = base + k * CHUNK
            pltpu.async_copy(gb, g_hbm.at[pl.ds(off, CHUNK)], semw)
            pltpu.async_copy(hb, h_hbm.at[pl.ds(off, CHUNK)], semw)
            pltpu.async_copy(db, xd_hbm.at[pl.ds(off * XW, CHUNK * XW)], semw)

        def drain(gb, hb, db, semw):
            pltpu.make_async_copy(g_hbm.at[pl.ds(0, CHUNK)], gb, semw).wait()
            pltpu.make_async_copy(h_hbm.at[pl.ds(0, CHUNK)], hb, semw).wait()
            pltpu.make_async_copy(
                xd_hbm.at[pl.ds(0, CHUNK * XW)], db, semw).wait()

        def body(k2, carry):
            ka = k2 * 2
            kb = ka + 1

            @pl.when(k2 > 0)
            def _():
                drain(gb0, hb0, db0, semw0)

            ca1, ca2 = issue(ka, gb0, hb0)

            @pl.when(k2 > 0)
            def _():
                drain(gb1, hb1, db1, semw1)

            cb1, cb2 = issue(kb, gb1, hb1)
            xcompute(ka, db0)
            xcompute(kb, db1)
            ca1.wait()
            ca2.wait()
            write(ka, gb0, hb0, db0, semw0)
            cb1.wait()
            cb2.wait()
            write(kb, gb1, hb1, db1, semw1)
            return carry

        lax.fori_loop(0, NCHUNK // 2, body, 0)
        drain(gb0, hb0, db0, semw0)
        drain(gb1, hb1, db1, semw1)

    @functools.partial(
        pl.kernel,
        mesh=mesh,
        out_type=(
            jax.ShapeDtypeStruct((NC, NPAD, CH), jnp.float32),
            jax.ShapeDtypeStruct((NW, XF), jnp.float32),
        ),
        scratch_types=(
            pltpu.VMEM((CHUNK,), jnp.int32),
            pltpu.VMEM((CHUNK,), jnp.int32),
            pltpu.VMEM((CHUNK, CH), jnp.float32),
            pltpu.VMEM((CHUNK * XW,), jnp.float32),
            pltpu.VMEM((CHUNK * XW,), jnp.float32),
            pltpu.VMEM((XF,), jnp.float32),
            pltpu.VMEM_SHARED((NPAD, CH), jnp.float32),
            pltpu.SemaphoreType.DMA,
        ),
        compiler_params=pltpu.CompilerParams(needs_layout_passes=False),
    )
    def _sc_scatter(m2_hbm, msg_hbm, dst_hbm,
                    hn_hbm, xn_hbm,
                    idv0, idv1, mb, msb0, msb1, xntab, hn_sp, seml):
        cid = lax.axis_index("c")
        sid = lax.axis_index("s")
        wid = sid * NC + cid
        base = wid * EPW
        iota16 = lax.iota(jnp.int32, 16)
        zero16 = jnp.zeros((16,), jnp.float32)

        def zrow(i, carry):
            def zcol(j, c2):
                mb[i, pl.ds(j * 16, 16)] = zero16
                return c2
            lax.fori_loop(0, CH // 16, zcol, 0)
            return carry

        lax.fori_loop(0, CHUNK, zrow, 0)

        def zx(i, carry):
            xntab[pl.ds(i * 16, 16)] = zero16
            return carry
        lax.fori_loop(0, XF // 16, zx, 0)

        for t in range(RPT // CHUNK):
            pltpu.sync_copy(mb, hn_sp.at[pl.ds(sid * RPT + t * CHUNK, CHUNK)])

        def load(k, idv, msb):
            off = base + k * CHUNK
            pltpu.async_copy(dst_hbm.at[pl.ds(off, CHUNK)], idv, seml)
            pltpu.async_copy(
                msg_hbm.at[pl.ds(off * XW, CHUNK * XW)], msb, seml)
            pltpu.async_copy(m2_hbm.at[pl.ds(off, CHUNK)], mb, seml)

        def drain_loads(idv, msb):
            pltpu.make_async_copy(
                dst_hbm.at[pl.ds(0, CHUNK)], idv, seml).wait()
            pltpu.make_async_copy(
                msg_hbm.at[pl.ds(0, CHUNK * XW)], msb, seml).wait()
            pltpu.make_async_copy(
                m2_hbm.at[pl.ds(0, CHUNK)], mb, seml).wait()

        def xncompute(idv, msb):
            for g in range(CHUNK // 16):
                rd = idv[pl.ds(g * 16, 16)] * 3
                lane = (iota16 + g * 16) * XW
                for j in range(3):
                    vals = plsc.load_gather(msb, [lane + j])
                    plsc.addupdate_scatter(xntab, [rd + j], vals)

        load(0, idv0, msb0)
        plsc.subcore_barrier()

        def body(k2, carry):
            ka = k2 * 2
            kb = ka + 1
            drain_loads(idv0, msb0)
            pltpu.sync_copy(mb, hn_sp.at[idv0], add=True)
            load(kb, idv1, msb1)
            xncompute(idv0, msb0)
            drain_loads(idv1, msb1)
            pltpu.sync_copy(mb, hn_sp.at[idv1], add=True)

            @pl.when(k2 < NCHUNK // 2 - 1)
            def _():
                load(ka + 2, idv0, msb0)

            xncompute(idv1, msb1)
            return carry

        lax.fori_loop(0, NCHUNK // 2, body, 0)
        plsc.subcore_barrier()
        pltpu.sync_copy(hn_sp.at[pl.ds(sid * RPT, RPT)],
                        hn_hbm.at[cid, pl.ds(sid * RPT, RPT)])
        pltpu.sync_copy(xntab, xn_hbm.at[wid])

    _sc_cache['gather'] = _sc_gather
    _sc_cache['scatter'] = _sc_scatter
    return _sc_gather, _sc_scatter


def _pre_body(h_ref, at_ref, bt_ref, pk_ref, p_ref, q_ref):
    h = h_ref[...]
    p_ref[...] = _mm(h, at_ref[...]).astype(jnp.bfloat16)
    q_ref[...] = (_mm(h, bt_ref[...]) + pk_ref[0:1, :]).astype(jnp.bfloat16)


def _edge_body(g_ref, h_ref, xd_ref, w1_ref, w2_ref, pk_ref,
               m2_ref, msg_ref):
    d = xd_ref[...]
    radial = d[:, 3:4]
    rn = 1.0 / (jnp.sqrt(radial) + 1e-30)
    mask = (lax.broadcasted_iota(jnp.int32, (1, XW), 1) < 3
            ).astype(jnp.float32)
    dn = d * rn * mask
    m1 = _silu(g_ref[...].astype(jnp.float32) + h_ref[...].astype(jnp.float32)
               + radial * pk_ref[3:4, :])
    m2 = _silu(_mm(m1, w1_ref[...]) + pk_ref[0:1, :])
    m2_ref[...] = m2
    cp = _silu(_mm(m2, w2_ref[...]) + pk_ref[1:2, :])
    scal = jnp.sum(cp * pk_ref[2:3, :], axis=1, keepdims=True)
    msg_ref[...] = scal * dn


def _node_body(h_ref, hn_ref, x_ref, xn_ref, ct_ref, dt_ref, w1t_ref,
               at_ref, bt_ref, pk_ref,
               hout_ref, xout_ref, p_ref, q_ref, qf_ref):
    hn = hn_ref[0] + hn_ref[1]
    h = h_ref[...]
    u = _silu(_mm(h, ct_ref[...]) + _mm(hn, dt_ref[...]) + pk_ref[0:1, :])
    v = _mm(u, w1t_ref[...]) + pk_ref[1:2, :]
    g = 0.5 * v * (1.0 + lax.erf(v * 0.7071067811865476))
    mu = jnp.mean(g, axis=1, keepdims=True)
    var = jnp.mean((g - mu) ** 2, axis=1, keepdims=True)
    hnew = (g - mu) * lax.rsqrt(var + 1e-5) * pk_ref[2:3, :] + pk_ref[3:4, :]
    hout_ref[...] = hnew
    xout_ref[...] = x_ref[...] + jnp.sum(xn_ref[...], axis=0)
    p_ref[...] = _mm(hnew, at_ref[...]).astype(jnp.bfloat16)
    q = _mm(hnew, bt_ref[...]) + pk_ref[4:5, :]
    q_ref[...] = q.astype(jnp.bfloat16)
    qf_ref[...] = q


def _wspec():
    return pl.BlockSpec((CH, CH), lambda i: (0, 0))


def _pkspec():
    return pl.BlockSpec((8, CH), lambda i: (0, 0))


def _tc_pre(h, at, bt, pk):
    return pl.pallas_call(
        _pre_body,
        grid=(NPAD // BN,),
        in_specs=[
            pl.BlockSpec((BN, CH), lambda i: (i, 0)),
            _wspec(), _wspec(), _pkspec(),
        ],
        out_specs=[
            pl.BlockSpec((BN, CH), lambda i: (i, 0)),
            pl.BlockSpec((BN, CH), lambda i: (i, 0)),
        ],
        out_shape=[
            jax.ShapeDtypeStruct((NPAD, CH), jnp.bfloat16),
            jax.ShapeDtypeStruct((NPAD, CH), jnp.bfloat16),
        ],
    )(h, at, bt, pk)


def _tc_edge(g, h, xd, w1, w2, pk):
    return pl.pallas_call(
        _edge_body,
        grid=(EP // BE,),
        in_specs=[
            pl.BlockSpec((BE, CH), lambda i: (i, 0)),
            pl.BlockSpec((BE, CH), lambda i: (i, 0)),
            pl.BlockSpec((BE, XW), lambda i: (i, 0)),
            _wspec(), _wspec(), _pkspec(),
        ],
        out_specs=[
            pl.BlockSpec((BE, CH), lambda i: (i, 0)),
            pl.BlockSpec((BE, XW), lambda i: (i, 0)),
        ],
        out_shape=[
            jax.ShapeDtypeStruct((EP, CH), jnp.float32),
            jax.ShapeDtypeStruct((EP, XW), jnp.float32),
        ],
    )(g, h, xd, w1, w2, pk)


def _tc_node(h, hn, x, xn, ct, dt, w1t, at, bt, pk):
    return pl.pallas_call(
        _node_body,
        grid=(NPAD // BN,),
        in_specs=[
            pl.BlockSpec((BN, CH), lambda i: (i, 0)),
            pl.BlockSpec((NC, BN, CH), lambda i: (0, i, 0)),
            pl.BlockSpec((BN, 3), lambda i: (i, 0)),
            pl.BlockSpec((NW, BN, 3), lambda i: (0, i, 0)),
            _wspec(), _wspec(), _wspec(), _wspec(), _wspec(), _pkspec(),
        ],
        out_specs=[
            pl.BlockSpec((BN, CH), lambda i: (i, 0)),
            pl.BlockSpec((BN, 3), lambda i: (i, 0)),
            pl.BlockSpec((BN, CH), lambda i: (i, 0)),
            pl.BlockSpec((BN, CH), lambda i: (i, 0)),
            pl.BlockSpec((BN, CH), lambda i: (i, 0)),
        ],
        out_shape=[
            jax.ShapeDtypeStruct((NPAD, CH), jnp.float32),
            jax.ShapeDtypeStruct((NPAD, 3), jnp.float32),
            jax.ShapeDtypeStruct((NPAD, CH), jnp.bfloat16),
            jax.ShapeDtypeStruct((NPAD, CH), jnp.bfloat16),
            jax.ShapeDtypeStruct((NPAD, CH), jnp.float32),
        ],
    )(h, hn, x, xn, ct, dt, w1t, at, bt, pk)


def _pack(rows):
    pk = jnp.zeros((8, CH), jnp.float32)
    for r, v in enumerate(rows):
        pk = pk.at[r, :].set(v)
    return pk


def kernel(node_feat, xyz, edge_index, params):
    h = jnp.zeros((NPAD, CH), jnp.float32).at[:N].set(node_feat)
    x = jnp.zeros((NPAD, 3), jnp.float32).at[:N].set(xyz)
    src = jnp.full((EP,), NPAD - 1, jnp.int32).at[:E].set(edge_index[0])
    dst = jnp.full((EP,), NPAD - 1, jnp.int32).at[:E].set(edge_index[1])

    lp = [params['layer%d' % i] for i in range(2)]
    AT = [p['We0'][:, :CH].T for p in lp]
    BT = [p['We0'][:, CH:2 * CH].T for p in lp]
    WR = [p['We0'][:, 2 * CH] for p in lp]
    pk_edge = [_pack([p['be1'], p['bc0'], p['Wc1'][0], wr])
               for p, wr in zip(lp, WR)]
    W1T = [p['We1'].T for p in lp]
    WC0T = [p['Wc0'].T for p in lp]
    CT = [p['Wn0'][:, :CH].T for p in lp]
    DT = [p['Wn0'][:, CH:].T for p in lp]
    WN1T = [p['Wn1'].T for p in lp]
    WOT = params['Wout'].T

    pk_pre = _pack([lp[0]['be0']])
    pk_node = [
        _pack([lp[0]['bn0'], lp[0]['bn1'], params['ln_g'], params['ln_b'],
               lp[1]['be0']]),
        _pack([lp[1]['bn0'], lp[1]['bn1'], params['ln_g'], params['ln_b'],
               params['bout']]),
    ]
    nxt_at = [AT[1], WOT]
    nxt_bt = [BT[1], WOT]

    _sc_gather, _sc_scatter = _build_sc_kernels()

    def _as_i32(a):
        return lax.bitcast_convert_type(
            a.reshape(NPAD, CH // 2, 2), jnp.int32)

    def _as_bf16(a):
        return lax.bitcast_convert_type(a, jnp.bfloat16).reshape(EP, CH)

    P, Q = _tc_pre(h, AT[0], BT[0], pk_pre)
    for i in range(2):
        G, H, XDf = _sc_gather(_as_i32(P), _as_i32(Q), x.reshape(XF),
                               src, dst)
        m2, msg = _tc_edge(_as_bf16(G), _as_bf16(H), XDf.reshape(EP, XW),
                           W1T[i], WC0T[i], pk_edge[i])
        hn, xn = _sc_scatter(m2, msg.reshape(EP * XW), dst)
        h, x, P, Q, Qf = _tc_node(h, hn, x, xn.reshape(NW, NPAD, 3),
                                  CT[i], DT[i], WN1T[i],
                                  nxt_at[i], nxt_bt[i], pk_node[i])
    return Qf[:N]


# speedup vs baseline: 1.3805x; 1.3805x over previous
import functools

import jax
import jax.numpy as jnp
from jax import lax
from jax.experimental import pallas as pl
from jax.experimental.pallas import tpu as pltpu
from jax.experimental.pallas import tpu_sc as plsc

N = 10000
E = 320000
CH = 128
NPAD = 10240
EP = 327680
XW = 4
NC = 2
NS = 16
NW = NC * NS
EPW = EP // NW
CHUNK = 128
NCHUNK = EPW // CHUNK
RPT = NPAD // NS
XF = NPAD * 3

BE = 2048
BN = 1024


def _silu(x):
    return x * jax.nn.sigmoid(x)


def _mm(a, b):
    return jnp.dot(a, b, preferred_element_type=jnp.float32)


_sc_cache = {}


def _build_sc_kernels():
    if _sc_cache:
        return _sc_cache['gather'], _sc_cache['scatter']

    mesh = plsc.VectorSubcoreMesh(core_axis_name="c", subcore_axis_name="s")

    @functools.partial(
        pl.kernel,
        mesh=mesh,
        out_type=(
            jax.ShapeDtypeStruct((EP, CH), jnp.float32),
            jax.ShapeDtypeStruct((EP * XW,), jnp.float32),
        ),
        scratch_types=(
            pltpu.VMEM((EPW,), jnp.int32),
            pltpu.VMEM((EPW,), jnp.int32),
            pltpu.VMEM((CHUNK, CH), jnp.float32),
            pltpu.VMEM((CHUNK, CH), jnp.float32),
            pltpu.VMEM((CHUNK * XW,), jnp.float32),
            pltpu.VMEM((CHUNK, CH), jnp.float32),
            pltpu.VMEM((CHUNK, CH), jnp.float32),
            pltpu.VMEM((CHUNK * XW,), jnp.float32),
            pltpu.VMEM((XF,), jnp.float32),
            pltpu.VMEM((CH,), jnp.float32),
            pltpu.SemaphoreType.DMA,
            pltpu.SemaphoreType.DMA,
            pltpu.SemaphoreType.DMA,
        ),
        compiler_params=pltpu.CompilerParams(needs_layout_passes=False),
    )
    def _sc_gather(p_hbm, q_hbm, x_hbm, wr_hbm, src_hbm, dst_hbm,
                   m1_hbm, xd_hbm,
                   s_all, d_all, gb0, hb0, db0, gb1, hb1, db1, xtab, wrt,
                   semg, semw0, semw1):
        wid = lax.axis_index("s") * NC + lax.axis_index("c")
        base = wid * EPW
        iota16 = lax.iota(jnp.int32, 16)
        pltpu.sync_copy(x_hbm, xtab)
        pltpu.sync_copy(wr_hbm, wrt)
        pltpu.sync_copy(src_hbm.at[pl.ds(base, EPW)], s_all)
        pltpu.sync_copy(dst_hbm.at[pl.ds(base, EPW)], d_all)
        wrv = [wrt[pl.ds(v * 16, 16)] for v in range(CH // 16)]

        def issue(k, gb, hb):
            sl = pl.ds(k * CHUNK, CHUNK)
            c1 = pltpu.async_copy(p_hbm.at[s_all.at[sl]], gb, semg)
            c2 = pltpu.async_copy(q_hbm.at[d_all.at[sl]], hb, semg)
            return c1, c2

        def xcompute(k, db):
            for g in range(CHUNK // 16):
                o = k * CHUNK + g * 16
                rs = s_all[pl.ds(o, 16)] * 3
                rd = d_all[pl.ds(o, 16)] * 3
                dx = (plsc.load_gather(xtab, [rs]) -
                      plsc.load_gather(xtab, [rd]))
                dy = (plsc.load_gather(xtab, [rs + 1]) -
                      plsc.load_gather(xtab, [rd + 1]))
                dz_ = (plsc.load_gather(xtab, [rs + 2]) -
                       plsc.load_gather(xtab, [rd + 2]))
                radial = dx * dx + dy * dy + dz_ * dz_
                row = (iota16 + g * 16) * XW
                plsc.store_scatter(db, [row], dx)
                plsc.store_scatter(db, [row + 1], dy)
                plsc.store_scatter(db, [row + 2], dz_)
                plsc.store_scatter(db, [row + 3], radial)

        def m1compute(gb, hb, db):
            def grp(g, carry):
                for i in range(16):
                    e = g * 16 + i
                    rsp = plsc.load_gather(
                        db, [jnp.zeros((16,), jnp.int32) + (e * XW + 3)])
                    for v in range(CH // 16):
                        sl = pl.ds(v * 16, 16)
                        val = gb[e, sl] + hb[e, sl] + rsp * wrv[v]
                        gb[e, sl] = val / (1.0 + jnp.exp(-val))
                return carry
            lax.fori_loop(0, CHUNK // 16, grp, 0)

        def write(k, gb, db, semw):
            off = base + k * CHUNK
            pltpu.async_copy(gb, m1_hbm.at[pl.ds(off, CHUNK)], semw)
            pltpu.async_copy(db, xd_hbm.at[pl.ds(off * XW, CHUNK * XW)], semw)

        def drain(gb, db, semw):
            pltpu.make_async_copy(m1_hbm.at[pl.ds(0, CHUNK)], gb, semw).wait()
            pltpu.make_async_copy(
                xd_hbm.at[pl.ds(0, CHUNK * XW)], db, semw).wait()

        def body(k2, carry):
            ka = k2 * 2
            kb = ka + 1

            @pl.when(k2 > 0)
            def _():
                drain(gb0, db0, semw0)

            ca1, ca2 = issue(ka, gb0, hb0)

            @pl.when(k2 > 0)
            def _():
                drain(gb1, db1, semw1)

            cb1, cb2 = issue(kb, gb1, hb1)
            xcompute(ka, db0)
            xcompute(kb, db1)
            ca1.wait()
            ca2.wait()
            m1compute(gb0, hb0, db0)
            write(ka, gb0, db0, semw0)
            cb1.wait()
            cb2.wait()
            m1compute(gb1, hb1, db1)
            write(kb, gb1, db1, semw1)
            return carry

        lax.fori_loop(0, NCHUNK // 2, body, 0)
        drain(gb0, db0, semw0)
        drain(gb1, db1, semw1)

    @functools.partial(
        pl.kernel,
        mesh=mesh,
        out_type=(
            jax.ShapeDtypeStruct((NC, NPAD, CH), jnp.float32),
            jax.ShapeDtypeStruct((NW, XF), jnp.float32),
        ),
        scratch_types=(
            pltpu.VMEM((CHUNK,), jnp.int32),
            pltpu.VMEM((CHUNK,), jnp.int32),
            pltpu.VMEM((CHUNK, CH), jnp.float32),
            pltpu.VMEM((CHUNK * XW,), jnp.float32),
            pltpu.VMEM((CHUNK * XW,), jnp.float32),
            pltpu.VMEM((XF,), jnp.float32),
            pltpu.VMEM_SHARED((NPAD, CH), jnp.float32),
            pltpu.SemaphoreType.DMA,
        ),
        compiler_params=pltpu.CompilerParams(needs_layout_passes=False),
    )
    def _sc_scatter(m2_hbm, msg_hbm, dst_hbm,
                    hn_hbm, xn_hbm,
                    idv0, idv1, mb, msb0, msb1, xntab, hn_sp, seml):
        cid = lax.axis_index("c")
        sid = lax.axis_index("s")
        wid = sid * NC + cid
        base = wid * EPW
        iota16 = lax.iota(jnp.int32, 16)
        zero16 = jnp.zeros((16,), jnp.float32)

        def zrow(i, carry):
            def zcol(j, c2):
                mb[i, pl.ds(j * 16, 16)] = zero16
                return c2
            lax.fori_loop(0, CH // 16, zcol, 0)
            return carry

        lax.fori_loop(0, CHUNK, zrow, 0)

        def zx(i, carry):
            xntab[pl.ds(i * 16, 16)] = zero16
            return carry
        lax.fori_loop(0, XF // 16, zx, 0)

        for t in range(RPT // CHUNK):
            pltpu.sync_copy(mb, hn_sp.at[pl.ds(sid * RPT + t * CHUNK, CHUNK)])

        def load(k, idv, msb):
            off = base + k * CHUNK
            pltpu.async_copy(dst_hbm.at[pl.ds(off, CHUNK)], idv, seml)
            pltpu.async_copy(
                msg_hbm.at[pl.ds(off * XW, CHUNK * XW)], msb, seml)
            pltpu.async_copy(m2_hbm.at[pl.ds(off, CHUNK)], mb, seml)

        def drain_loads(idv, msb):
            pltpu.make_async_copy(
                dst_hbm.at[pl.ds(0, CHUNK)], idv, seml).wait()
            pltpu.make_async_copy(
                msg_hbm.at[pl.ds(0, CHUNK * XW)], msb, seml).wait()
            pltpu.make_async_copy(
                m2_hbm.at[pl.ds(0, CHUNK)], mb, seml).wait()

        def xncompute(idv, msb):
            for g in range(CHUNK // 16):
                rd = idv[pl.ds(g * 16, 16)] * 3
                lane = (iota16 + g * 16) * XW
                for j in range(3):
                    vals = plsc.load_gather(msb, [lane + j])
                    plsc.addupdate_scatter(xntab, [rd + j], vals)

        load(0, idv0, msb0)
        plsc.subcore_barrier()

        def body(k2, carry):
            ka = k2 * 2
            kb = ka + 1
            drain_loads(idv0, msb0)
            pltpu.sync_copy(mb, hn_sp.at[idv0], add=True)
            load(kb, idv1, msb1)
            xncompute(idv0, msb0)
            drain_loads(idv1, msb1)
            pltpu.sync_copy(mb, hn_sp.at[idv1], add=True)

            @pl.when(k2 < NCHUNK // 2 - 1)
            def _():
                load(ka + 2, idv0, msb0)

            xncompute(idv1, msb1)
            return carry

        lax.fori_loop(0, NCHUNK // 2, body, 0)
        plsc.subcore_barrier()
        pltpu.sync_copy(hn_sp.at[pl.ds(sid * RPT, RPT)],
                        hn_hbm.at[cid, pl.ds(sid * RPT, RPT)])
        pltpu.sync_copy(xntab, xn_hbm.at[wid])

    _sc_cache['gather'] = _sc_gather
    _sc_cache['scatter'] = _sc_scatter
    return _sc_gather, _sc_scatter


def _pre_body(h_ref, at_ref, bt_ref, pk_ref, p_ref, q_ref):
    h = h_ref[...]
    p_ref[...] = _mm(h, at_ref[...])
    q_ref[...] = _mm(h, bt_ref[...]) + pk_ref[0:1, :]


def _edge_body(m1_ref, xd_ref, w1_ref, w2_ref, pk_ref,
               m2_ref, msg_ref):
    d = xd_ref[...]
    radial = d[:, 3:4]
    rn = 1.0 / (jnp.sqrt(radial) + 1e-30)
    mask = (lax.broadcasted_iota(jnp.int32, (1, XW), 1) < 3
            ).astype(jnp.float32)
    dn = d * rn * mask
    m2 = _silu(_mm(m1_ref[...], w1_ref[...]) + pk_ref[0:1, :])
    m2_ref[...] = m2
    cp = _silu(_mm(m2, w2_ref[...]) + pk_ref[1:2, :])
    scal = jnp.sum(cp * pk_ref[2:3, :], axis=1, keepdims=True)
    msg_ref[...] = scal * dn


def _node_body(h_ref, hn_ref, x_ref, xn_ref, ct_ref, dt_ref, w1t_ref,
               at_ref, bt_ref, pk_ref,
               hout_ref, xout_ref, p_ref, q_ref):
    hn = hn_ref[0] + hn_ref[1]
    h = h_ref[...]
    u = _silu(_mm(h, ct_ref[...]) + _mm(hn, dt_ref[...]) + pk_ref[0:1, :])
    v = _mm(u, w1t_ref[...]) + pk_ref[1:2, :]
    g = 0.5 * v * (1.0 + lax.erf(v * 0.7071067811865476))
    mu = jnp.mean(g, axis=1, keepdims=True)
    var = jnp.mean((g - mu) ** 2, axis=1, keepdims=True)
    hnew = (g - mu) * lax.rsqrt(var + 1e-5) * pk_ref[2:3, :] + pk_ref[3:4, :]
    hout_ref[...] = hnew
    xout_ref[...] = x_ref[...] + jnp.sum(xn_ref[...], axis=0)
    p_ref[...] = _mm(hnew, at_ref[...])
    q_ref[...] = _mm(hnew, bt_ref[...]) + pk_ref[4:5, :]


def _wspec():
    return pl.BlockSpec((CH, CH), lambda i: (0, 0))


def _pkspec():
    return pl.BlockSpec((8, CH), lambda i: (0, 0))


def _tc_pre(h, at, bt, pk):
    return pl.pallas_call(
        _pre_body,
        grid=(NPAD // BN,),
        in_specs=[
            pl.BlockSpec((BN, CH), lambda i: (i, 0)),
            _wspec(), _wspec(), _pkspec(),
        ],
        out_specs=[
            pl.BlockSpec((BN, CH), lambda i: (i, 0)),
            pl.BlockSpec((BN, CH), lambda i: (i, 0)),
        ],
        out_shape=[
            jax.ShapeDtypeStruct((NPAD, CH), jnp.float32),
            jax.ShapeDtypeStruct((NPAD, CH), jnp.float32),
        ],
    )(h, at, bt, pk)


def _tc_edge(m1, xd, w1, w2, pk):
    return pl.pallas_call(
        _edge_body,
        grid=(EP // BE,),
        in_specs=[
            pl.BlockSpec((BE, CH), lambda i: (i, 0)),
            pl.BlockSpec((BE, XW), lambda i: (i, 0)),
            _wspec(), _wspec(), _pkspec(),
        ],
        out_specs=[
            pl.BlockSpec((BE, CH), lambda i: (i, 0)),
            pl.BlockSpec((BE, XW), lambda i: (i, 0)),
        ],
        out_shape=[
            jax.ShapeDtypeStruct((EP, CH), jnp.float32),
            jax.ShapeDtypeStruct((EP, XW), jnp.float32),
        ],
    )(m1, xd, w1, w2, pk)


def _tc_node(h, hn, x, xn, ct, dt, w1t, at, bt, pk):
    return pl.pallas_call(
        _node_body,
        grid=(NPAD // BN,),
        in_specs=[
            pl.BlockSpec((BN, CH), lambda i: (i, 0)),
            pl.BlockSpec((NC, BN, CH), lambda i: (0, i, 0)),
            pl.BlockSpec((BN, 3), lambda i: (i, 0)),
            pl.BlockSpec((NW, BN, 3), lambda i: (0, i, 0)),
            _wspec(), _wspec(), _wspec(), _wspec(), _wspec(), _pkspec(),
        ],
        out_specs=[
            pl.BlockSpec((BN, CH), lambda i: (i, 0)),
            pl.BlockSpec((BN, 3), lambda i: (i, 0)),
            pl.BlockSpec((BN, CH), lambda i: (i, 0)),
            pl.BlockSpec((BN, CH), lambda i: (i, 0)),
        ],
        out_shape=[
            jax.ShapeDtypeStruct((NPAD, CH), jnp.float32),
            jax.ShapeDtypeStruct((NPAD, 3), jnp.float32),
            jax.ShapeDtypeStruct((NPAD, CH), jnp.float32),
            jax.ShapeDtypeStruct((NPAD, CH), jnp.float32),
        ],
    )(h, hn, x, xn, ct, dt, w1t, at, bt, pk)


def _pack(rows):
    pk = jnp.zeros((8, CH), jnp.float32)
    for r, v in enumerate(rows):
        pk = pk.at[r, :].set(v)
    return pk


def kernel(node_feat, xyz, edge_index, params):
    h = jnp.zeros((NPAD, CH), jnp.float32).at[:N].set(node_feat)
    x = jnp.zeros((NPAD, 3), jnp.float32).at[:N].set(xyz)
    src = jnp.full((EP,), NPAD - 1, jnp.int32).at[:E].set(edge_index[0])
    dst = jnp.full((EP,), NPAD - 1, jnp.int32).at[:E].set(edge_index[1])

    lp = [params['layer%d' % i] for i in range(2)]
    AT = [p['We0'][:, :CH].T for p in lp]
    BT = [p['We0'][:, CH:2 * CH].T for p in lp]
    WR = [p['We0'][:, 2 * CH] for p in lp]
    pk_edge = [_pack([p['be1'], p['bc0'], p['Wc1'][0], wr])
               for p, wr in zip(lp, WR)]
    W1T = [p['We1'].T for p in lp]
    WC0T = [p['Wc0'].T for p in lp]
    CT = [p['Wn0'][:, :CH].T for p in lp]
    DT = [p['Wn0'][:, CH:].T for p in lp]
    WN1T = [p['Wn1'].T for p in lp]
    WOT = params['Wout'].T

    pk_pre = _pack([lp[0]['be0']])
    pk_node = [
        _pack([lp[0]['bn0'], lp[0]['bn1'], params['ln_g'], params['ln_b'],
               lp[1]['be0']]),
        _pack([lp[1]['bn0'], lp[1]['bn1'], params['ln_g'], params['ln_b'],
               params['bout']]),
    ]
    nxt_at = [AT[1], WOT]
    nxt_bt = [BT[1], WOT]

    _sc_gather, _sc_scatter = _build_sc_kernels()

    P, Q = _tc_pre(h, AT[0], BT[0], pk_pre)
    for i in range(2):
        M1, XDf = _sc_gather(P, Q, x.reshape(XF), WR[i], src, dst)
        m2, msg = _tc_edge(M1, XDf.reshape(EP, XW),
                           W1T[i], WC0T[i], pk_edge[i])
        hn, xn = _sc_scatter(m2, msg.reshape(EP * XW), dst)
        h, x, P, Q = _tc_node(h, hn, x, xn.reshape(NW, NPAD, 3),
                              CT[i], DT[i], WN1T[i],
                              nxt_at[i], nxt_bt[i], pk_node[i])
    return Q[:N]


# speedup vs baseline: 1.5878x; 1.1502x over previous
import functools

import jax
import jax.numpy as jnp
from jax import lax
from jax.experimental import pallas as pl
from jax.experimental.pallas import tpu as pltpu
from jax.experimental.pallas import tpu_sc as plsc

N = 10000
E = 320000
CH = 128
NPAD = 10240
EP = 327680
XW = 4
NC = 2
NS = 16
NW = NC * NS
EPW = EP // NW
CHUNK = 128
NCHUNK = EPW // CHUNK
RPT = NPAD // NS
XF = NPAD * 3

BE = 2048
BN = 1024


def _silu(x):
    return x * jax.nn.sigmoid(x)


def _mm(a, b):
    return jnp.dot(a, b, preferred_element_type=jnp.float32)


_sc_cache = {}


def _build_sc_kernels():
    if _sc_cache:
        return _sc_cache['gather'], _sc_cache['scatter']

    mesh = plsc.VectorSubcoreMesh(core_axis_name="c", subcore_axis_name="s")

    @functools.partial(
        pl.kernel,
        mesh=mesh,
        out_type=(
            jax.ShapeDtypeStruct((EP, CH), jnp.float32),
            jax.ShapeDtypeStruct((EP, CH), jnp.float32),
            jax.ShapeDtypeStruct((EP * XW,), jnp.float32),
        ),
        scratch_types=(
            pltpu.VMEM((EPW,), jnp.int32),
            pltpu.VMEM((EPW,), jnp.int32),
            pltpu.VMEM((CHUNK, CH), jnp.float32),
            pltpu.VMEM((CHUNK, CH), jnp.float32),
            pltpu.VMEM((CHUNK * XW,), jnp.float32),
            pltpu.VMEM((CHUNK, CH), jnp.float32),
            pltpu.VMEM((CHUNK, CH), jnp.float32),
            pltpu.VMEM((CHUNK * XW,), jnp.float32),
            pltpu.VMEM((XF,), jnp.float32),
            pltpu.SemaphoreType.DMA,
            pltpu.SemaphoreType.DMA,
            pltpu.SemaphoreType.DMA,
        ),
        compiler_params=pltpu.CompilerParams(needs_layout_passes=False),
    )
    def _sc_gather(p_hbm, q_hbm, x_hbm, src_hbm, dst_hbm,
                   g_hbm, h_hbm, xd_hbm,
                   s_all, d_all, gb0, hb0, db0, gb1, hb1, db1, xtab,
                   semg, semw0, semw1):
        wid = lax.axis_index("s") * NC + lax.axis_index("c")
        base = wid * EPW
        iota16 = lax.iota(jnp.int32, 16)
        pltpu.sync_copy(x_hbm, xtab)
        pltpu.sync_copy(src_hbm.at[pl.ds(base, EPW)], s_all)
        pltpu.sync_copy(dst_hbm.at[pl.ds(base, EPW)], d_all)

        def issue(k, gb, hb):
            sl = pl.ds(k * CHUNK, CHUNK)
            c1 = pltpu.async_copy(p_hbm.at[s_all.at[sl]], gb, semg)
            c2 = pltpu.async_copy(q_hbm.at[d_all.at[sl]], hb, semg)
            return c1, c2

        def xcompute(k, db):
            for g in range(CHUNK // 16):
                o = k * CHUNK + g * 16
                rs = s_all[pl.ds(o, 16)] * 3
                rd = d_all[pl.ds(o, 16)] * 3
                dx = (plsc.load_gather(xtab, [rs]) -
                      plsc.load_gather(xtab, [rd]))
                dy = (plsc.load_gather(xtab, [rs + 1]) -
                      plsc.load_gather(xtab, [rd + 1]))
                dz_ = (plsc.load_gather(xtab, [rs + 2]) -
                       plsc.load_gather(xtab, [rd + 2]))
                radial = dx * dx + dy * dy + dz_ * dz_
                row = (iota16 + g * 16) * XW
                plsc.store_scatter(db, [row], dx)
                plsc.store_scatter(db, [row + 1], dy)
                plsc.store_scatter(db, [row + 2], dz_)
                plsc.store_scatter(db, [row + 3], radial)

        def write(k, gb, hb, db, semw):
            off = base + k * CHUNK
            pltpu.async_copy(gb, g_hbm.at[pl.ds(off, CHUNK)], semw)
            pltpu.async_copy(hb, h_hbm.at[pl.ds(off, CHUNK)], semw)
            pltpu.async_copy(db, xd_hbm.at[pl.ds(off * XW, CHUNK * XW)], semw)

        def drain(gb, hb, db, semw):
            pltpu.make_async_copy(g_hbm.at[pl.ds(0, CHUNK)], gb, semw).wait()
            pltpu.make_async_copy(h_hbm.at[pl.ds(0, CHUNK)], hb, semw).wait()
            pltpu.make_async_copy(
                xd_hbm.at[pl.ds(0, CHUNK * XW)], db, semw).wait()

        def body(k2, carry):
            ka = k2 * 2
            kb = ka + 1

            @pl.when(k2 > 0)
            def _():
                drain(gb0, hb0, db0, semw0)

            ca1, ca2 = issue(ka, gb0, hb0)

            @pl.when(k2 > 0)
            def _():
                drain(gb1, hb1, db1, semw1)

            cb1, cb2 = issue(kb, gb1, hb1)
            xcompute(ka, db0)
            xcompute(kb, db1)
            ca1.wait()
            ca2.wait()
            write(ka, gb0, hb0, db0, semw0)
            cb1.wait()
            cb2.wait()
            write(kb, gb1, hb1, db1, semw1)
            return carry

        lax.fori_loop(0, NCHUNK // 2, body, 0)
        drain(gb0, hb0, db0, semw0)
        drain(gb1, hb1, db1, semw1)

    @functools.partial(
        pl.kernel,
        mesh=mesh,
        out_type=(
            jax.ShapeDtypeStruct((NC, NPAD, CH), jnp.float32),
            jax.ShapeDtypeStruct((NW, XF), jnp.float32),
        ),
        scratch_types=(
            pltpu.VMEM((CHUNK,), jnp.int32),
            pltpu.VMEM((CHUNK,), jnp.int32),
            pltpu.VMEM((CHUNK, CH), jnp.float32),
            pltpu.VMEM((CHUNK * XW,), jnp.float32),
            pltpu.VMEM((CHUNK * XW,), jnp.float32),
            pltpu.VMEM((XF,), jnp.float32),
            pltpu.VMEM_SHARED((NPAD, CH), jnp.float32),
            pltpu.SemaphoreType.DMA,
        ),
        compiler_params=pltpu.CompilerParams(needs_layout_passes=False),
    )
    def _sc_scatter(m2_hbm, msg_hbm, dst_hbm,
                    hn_hbm, xn_hbm,
                    idv0, idv1, mb, msb0, msb1, xntab, hn_sp, seml):
        cid = lax.axis_index("c")
        sid = lax.axis_index("s")
        wid = sid * NC + cid
        base = wid * EPW
        iota16 = lax.iota(jnp.int32, 16)
        zero16 = jnp.zeros((16,), jnp.float32)

        def zrow(i, carry):
            def zcol(j, c2):
                mb[i, pl.ds(j * 16, 16)] = zero16
                return c2
            lax.fori_loop(0, CH // 16, zcol, 0)
            return carry

        lax.fori_loop(0, CHUNK, zrow, 0)

        def zx(i, carry):
            xntab[pl.ds(i * 16, 16)] = zero16
            return carry
        lax.fori_loop(0, XF // 16, zx, 0)

        for t in range(RPT // CHUNK):
            pltpu.sync_copy(mb, hn_sp.at[pl.ds(sid * RPT + t * CHUNK, CHUNK)])

        def load(k, idv, msb):
            off = base + k * CHUNK
            pltpu.async_copy(dst_hbm.at[pl.ds(off, CHUNK)], idv, seml)
            pltpu.async_copy(
                msg_hbm.at[pl.ds(off * XW, CHUNK * XW)], msb, seml)
            pltpu.async_copy(m2_hbm.at[pl.ds(off, CHUNK)], mb, seml)

        def drain_loads(idv, msb):
            pltpu.make_async_copy(
                dst_hbm.at[pl.ds(0, CHUNK)], idv, seml).wait()
            pltpu.make_async_copy(
                msg_hbm.at[pl.ds(0, CHUNK * XW)], msb, seml).wait()
            pltpu.make_async_copy(
                m2_hbm.at[pl.ds(0, CHUNK)], mb, seml).wait()

        def xncompute(idv, msb):
            for g in range(CHUNK // 16):
                rd = idv[pl.ds(g * 16, 16)] * 3
                lane = (iota16 + g * 16) * XW
                for j in range(3):
                    vals = plsc.load_gather(msb, [lane + j])
                    plsc.addupdate_scatter(xntab, [rd + j], vals)

        load(0, idv0, msb0)
        plsc.subcore_barrier()

        def body(k2, carry):
            ka = k2 * 2
            kb = ka + 1
            drain_loads(idv0, msb0)
            pltpu.sync_copy(mb, hn_sp.at[idv0], add=True)
            load(kb, idv1, msb1)
            xncompute(idv0, msb0)
            drain_loads(idv1, msb1)
            pltpu.sync_copy(mb, hn_sp.at[idv1], add=True)

            @pl.when(k2 < NCHUNK // 2 - 1)
            def _():
                load(ka + 2, idv0, msb0)

            xncompute(idv1, msb1)
            return carry

        lax.fori_loop(0, NCHUNK // 2, body, 0)
        plsc.subcore_barrier()
        pltpu.sync_copy(hn_sp.at[pl.ds(sid * RPT, RPT)],
                        hn_hbm.at[cid, pl.ds(sid * RPT, RPT)])
        pltpu.sync_copy(xntab, xn_hbm.at[wid])

    _sc_cache['gather'] = _sc_gather
    _sc_cache['scatter'] = _sc_scatter
    return _sc_gather, _sc_scatter


def _pre_body(h_ref, at_ref, bt_ref, pk_ref, p_ref, q_ref):
    h = h_ref[...]
    p_ref[...] = _mm(h, at_ref[...])
    q_ref[...] = _mm(h, bt_ref[...]) + pk_ref[0:1, :]


def _edge_body(g_ref, h_ref, xd_ref, w1_ref, w2_ref, pk_ref,
               m2_ref, msg_ref):
    d = xd_ref[...]
    radial = d[:, 3:4]
    rn = 1.0 / (jnp.sqrt(radial) + 1e-30)
    mask = (lax.broadcasted_iota(jnp.int32, (1, XW), 1) < 3
            ).astype(jnp.float32)
    dn = d * rn * mask
    m1 = _silu(g_ref[...] + h_ref[...] + radial * pk_ref[3:4, :])
    m2 = _silu(_mm(m1, w1_ref[...]) + pk_ref[0:1, :])
    m2_ref[...] = m2
    cp = _silu(_mm(m2, w2_ref[...]) + pk_ref[1:2, :])
    scal = jnp.sum(cp * pk_ref[2:3, :], axis=1, keepdims=True)
    msg_ref[...] = scal * dn


def _node_body(h_ref, hn_ref, x_ref, xn_ref, ct_ref, dt_ref, w1t_ref,
               at_ref, bt_ref, pk_ref,
               hout_ref, xout_ref, p_ref, q_ref):
    hn = hn_ref[0] + hn_ref[1]
    h = h_ref[...]
    u = _silu(_mm(h, ct_ref[...]) + _mm(hn, dt_ref[...]) + pk_ref[0:1, :])
    v = _mm(u, w1t_ref[...]) + pk_ref[1:2, :]
    g = 0.5 * v * (1.0 + lax.erf(v * 0.7071067811865476))
    mu = jnp.mean(g, axis=1, keepdims=True)
    var = jnp.mean((g - mu) ** 2, axis=1, keepdims=True)
    hnew = (g - mu) * lax.rsqrt(var + 1e-5) * pk_ref[2:3, :] + pk_ref[3:4, :]
    hout_ref[...] = hnew
    xout_ref[...] = x_ref[...] + jnp.sum(xn_ref[...], axis=0)
    p_ref[...] = _mm(hnew, at_ref[...])
    q_ref[...] = _mm(hnew, bt_ref[...]) + pk_ref[4:5, :]


def _wspec():
    return pl.BlockSpec((CH, CH), lambda i: (0, 0))


def _pkspec():
    return pl.BlockSpec((8, CH), lambda i: (0, 0))


def _tc_pre(h, at, bt, pk):
    return pl.pallas_call(
        _pre_body,
        grid=(NPAD // BN,),
        in_specs=[
            pl.BlockSpec((BN, CH), lambda i: (i, 0)),
            _wspec(), _wspec(), _pkspec(),
        ],
        out_specs=[
            pl.BlockSpec((BN, CH), lambda i: (i, 0)),
            pl.BlockSpec((BN, CH), lambda i: (i, 0)),
        ],
        out_shape=[
            jax.ShapeDtypeStruct((NPAD, CH), jnp.float32),
            jax.ShapeDtypeStruct((NPAD, CH), jnp.float32),
        ],
    )(h, at, bt, pk)


def _tc_edge(g, h, xd, w1, w2, pk):
    return pl.pallas_call(
        _edge_body,
        grid=(EP // BE,),
        in_specs=[
            pl.BlockSpec((BE, CH), lambda i: (i, 0)),
            pl.BlockSpec((BE, CH), lambda i: (i, 0)),
            pl.BlockSpec((BE, XW), lambda i: (i, 0)),
            _wspec(), _wspec(), _pkspec(),
        ],
        out_specs=[
            pl.BlockSpec((BE, CH), lambda i: (i, 0)),
            pl.BlockSpec((BE, XW), lambda i: (i, 0)),
        ],
        out_shape=[
            jax.ShapeDtypeStruct((EP, CH), jnp.float32),
            jax.ShapeDtypeStruct((EP, XW), jnp.float32),
        ],
    )(g, h, xd, w1, w2, pk)


def _tc_node(h, hn, x, xn, ct, dt, w1t, at, bt, pk):
    return pl.pallas_call(
        _node_body,
        grid=(NPAD // BN,),
        in_specs=[
            pl.BlockSpec((BN, CH), lambda i: (i, 0)),
            pl.BlockSpec((NC, BN, CH), lambda i: (0, i, 0)),
            pl.BlockSpec((BN, 3), lambda i: (i, 0)),
            pl.BlockSpec((NW, BN, 3), lambda i: (0, i, 0)),
            _wspec(), _wspec(), _wspec(), _wspec(), _wspec(), _pkspec(),
        ],
        out_specs=[
            pl.BlockSpec((BN, CH), lambda i: (i, 0)),
            pl.BlockSpec((BN, 3), lambda i: (i, 0)),
            pl.BlockSpec((BN, CH), lambda i: (i, 0)),
            pl.BlockSpec((BN, CH), lambda i: (i, 0)),
        ],
        out_shape=[
            jax.ShapeDtypeStruct((NPAD, CH), jnp.float32),
            jax.ShapeDtypeStruct((NPAD, 3), jnp.float32),
            jax.ShapeDtypeStruct((NPAD, CH), jnp.float32),
            jax.ShapeDtypeStruct((NPAD, CH), jnp.float32),
        ],
    )(h, hn, x, xn, ct, dt, w1t, at, bt, pk)


def _pack(rows):
    pk = jnp.zeros((8, CH), jnp.float32)
    for r, v in enumerate(rows):
        pk = pk.at[r, :].set(v)
    return pk


def kernel(node_feat, xyz, edge_index, params):
    h = jnp.zeros((NPAD, CH), jnp.float32).at[:N].set(node_feat)
    x = jnp.zeros((NPAD, 3), jnp.float32).at[:N].set(xyz)
    src = jnp.full((EP,), NPAD - 1, jnp.int32).at[:E].set(edge_index[0])
    dst = jnp.full((EP,), NPAD - 1, jnp.int32).at[:E].set(edge_index[1])

    lp = [params['layer%d' % i] for i in range(2)]
    AT = [p['We0'][:, :CH].T for p in lp]
    BT = [p['We0'][:, CH:2 * CH].T for p in lp]
    WR = [p['We0'][:, 2 * CH] for p in lp]
    pk_edge = [_pack([p['be1'], p['bc0'], p['Wc1'][0], wr])
               for p, wr in zip(lp, WR)]
    W1T = [p['We1'].T for p in lp]
    WC0T = [p['Wc0'].T for p in lp]
    CT = [p['Wn0'][:, :CH].T for p in lp]
    DT = [p['Wn0'][:, CH:].T for p in lp]
    WN1T = [p['Wn1'].T for p in lp]
    WOT = params['Wout'].T

    pk_pre = _pack([lp[0]['be0']])
    pk_node = [
        _pack([lp[0]['bn0'], lp[0]['bn1'], params['ln_g'], params['ln_b'],
               lp[1]['be0']]),
        _pack([lp[1]['bn0'], lp[1]['bn1'], params['ln_g'], params['ln_b'],
               params['bout']]),
    ]
    nxt_at = [AT[1], WOT]
    nxt_bt = [BT[1], WOT]

    _sc_gather, _sc_scatter = _build_sc_kernels()

    P, Q = _tc_pre(h, AT[0], BT[0], pk_pre)
    for i in range(2):
        G, H, XDf = _sc_gather(P, Q, x.reshape(XF), src, dst)
        m2, msg = _tc_edge(G, H, XDf.reshape(EP, XW),
                           W1T[i], WC0T[i], pk_edge[i])
        hn, xn = _sc_scatter(m2, msg.reshape(EP * XW), dst)
        h, x, P, Q = _tc_node(h, hn, x, xn.reshape(NW, NPAD, 3),
                              CT[i], DT[i], WN1T[i],
                              nxt_at[i], nxt_bt[i], pk_node[i])
    return Q[:N]


# speedup vs baseline: 2.5685x; 1.6176x over previous
import functools

import jax
import jax.numpy as jnp
from jax import lax
from jax.experimental import pallas as pl
from jax.experimental.pallas import tpu as pltpu
from jax.experimental.pallas import tpu_sc as plsc

N = 10000
E = 320000
CH = 128
NPAD = 10240
EP = 327680
XW = 4
NC = 2
NS = 16
NW = NC * NS
EPW = EP // NW
CHUNK = 128
NCHUNK = EPW // CHUNK
RPT = NPAD // NS
XF = NPAD * 3

BE = 2048
BN = 1024


def _silu(x):
    return x * jax.nn.sigmoid(x)


def _mm(a, b):
    return jnp.dot(a, b, preferred_element_type=jnp.float32)


_sc_cache = {}


def _build_sc_kernels():
    if _sc_cache:
        return _sc_cache['gather'], _sc_cache['scatter']

    mesh = plsc.VectorSubcoreMesh(core_axis_name="c", subcore_axis_name="s")

    @functools.partial(
        pl.kernel,
        mesh=mesh,
        out_type=(
            jax.ShapeDtypeStruct((EP, CH), jnp.float32),
            jax.ShapeDtypeStruct((EP, CH), jnp.float32),
            jax.ShapeDtypeStruct((EP * XW,), jnp.float32),
        ),
        scratch_types=(
            pltpu.VMEM((EPW,), jnp.int32),
            pltpu.VMEM((EPW,), jnp.int32),
            pltpu.VMEM((CHUNK, CH), jnp.float32),
            pltpu.VMEM((CHUNK, CH), jnp.float32),
            pltpu.VMEM((CHUNK * XW,), jnp.float32),
            pltpu.VMEM((CHUNK, CH), jnp.float32),
            pltpu.VMEM((CHUNK, CH), jnp.float32),
            pltpu.VMEM((CHUNK * XW,), jnp.float32),
            pltpu.VMEM((XF,), jnp.float32),
            pltpu.SemaphoreType.DMA,
            pltpu.SemaphoreType.DMA,
            pltpu.SemaphoreType.DMA,
        ),
        compiler_params=pltpu.CompilerParams(needs_layout_passes=False),
    )
    def _sc_gather(p_hbm, q_hbm, x_hbm, src_hbm, dst_hbm,
                   g_hbm, h_hbm, xd_hbm,
                   s_all, d_all, gb0, hb0, db0, gb1, hb1, db1, xtab,
                   semg, semw0, semw1):
        wid = lax.axis_index("s") * NC + lax.axis_index("c")
        base = wid * EPW
        iota16 = lax.iota(jnp.int32, 16)
        pltpu.sync_copy(x_hbm, xtab)
        pltpu.sync_copy(src_hbm.at[pl.ds(base, EPW)], s_all)
        pltpu.sync_copy(dst_hbm.at[pl.ds(base, EPW)], d_all)

        def issue(k, gb, hb):
            sl = pl.ds(k * CHUNK, CHUNK)
            c1 = pltpu.async_copy(p_hbm.at[s_all.at[sl]], gb, semg)
            c2 = pltpu.async_copy(q_hbm.at[d_all.at[sl]], hb, semg)
            return c1, c2

        def xcompute(k, db):
            for g in range(CHUNK // 16):
                o = k * CHUNK + g * 16
                rs = s_all[pl.ds(o, 16)] * 3
                rd = d_all[pl.ds(o, 16)] * 3
                dx = (plsc.load_gather(xtab, [rs]) -
                      plsc.load_gather(xtab, [rd]))
                dy = (plsc.load_gather(xtab, [rs + 1]) -
                      plsc.load_gather(xtab, [rd + 1]))
                dz_ = (plsc.load_gather(xtab, [rs + 2]) -
                       plsc.load_gather(xtab, [rd + 2]))
                radial = dx * dx + dy * dy + dz_ * dz_
                db[pl.ds(g * 16, 16)] = dx
                db[pl.ds(CHUNK + g * 16, 16)] = dy
                db[pl.ds(2 * CHUNK + g * 16, 16)] = dz_
                db[pl.ds(3 * CHUNK + g * 16, 16)] = radial

        def write(k, gb, hb, db, semw):
            off = base + k * CHUNK
            pltpu.async_copy(gb, g_hbm.at[pl.ds(off, CHUNK)], semw)
            pltpu.async_copy(hb, h_hbm.at[pl.ds(off, CHUNK)], semw)
            pltpu.async_copy(db, xd_hbm.at[pl.ds(off * XW, CHUNK * XW)], semw)

        def drain(gb, hb, db, semw):
            pltpu.make_async_copy(g_hbm.at[pl.ds(0, CHUNK)], gb, semw).wait()
            pltpu.make_async_copy(h_hbm.at[pl.ds(0, CHUNK)], hb, semw).wait()
            pltpu.make_async_copy(
                xd_hbm.at[pl.ds(0, CHUNK * XW)], db, semw).wait()

        def body(k2, carry):
            ka = k2 * 2
            kb = ka + 1

            @pl.when(k2 > 0)
            def _():
                drain(gb0, hb0, db0, semw0)

            ca1, ca2 = issue(ka, gb0, hb0)

            @pl.when(k2 > 0)
            def _():
                drain(gb1, hb1, db1, semw1)

            cb1, cb2 = issue(kb, gb1, hb1)
            xcompute(ka, db0)
            xcompute(kb, db1)
            ca1.wait()
            ca2.wait()
            write(ka, gb0, hb0, db0, semw0)
            cb1.wait()
            cb2.wait()
            write(kb, gb1, hb1, db1, semw1)
            return carry

        lax.fori_loop(0, NCHUNK // 2, body, 0)
        drain(gb0, hb0, db0, semw0)
        drain(gb1, hb1, db1, semw1)

    @functools.partial(
        pl.kernel,
        mesh=mesh,
        out_type=(
            jax.ShapeDtypeStruct((NC, NPAD, CH), jnp.float32),
            jax.ShapeDtypeStruct((NW, XF), jnp.float32),
        ),
        scratch_types=(
            pltpu.VMEM((CHUNK,), jnp.int32),
            pltpu.VMEM((CHUNK,), jnp.int32),
            pltpu.VMEM((CHUNK, CH), jnp.float32),
            pltpu.VMEM((CHUNK * XW,), jnp.float32),
            pltpu.VMEM((CHUNK * XW,), jnp.float32),
            pltpu.VMEM((XF,), jnp.float32),
            pltpu.VMEM_SHARED((NPAD, CH), jnp.float32),
            pltpu.SemaphoreType.DMA,
        ),
        compiler_params=pltpu.CompilerParams(needs_layout_passes=False),
    )
    def _sc_scatter(m2_hbm, msg_hbm, dst_hbm,
                    hn_hbm, xn_hbm,
                    idv0, idv1, mb, msb0, msb1, xntab, hn_sp, seml):
        cid = lax.axis_index("c")
        sid = lax.axis_index("s")
        wid = sid * NC + cid
        base = wid * EPW
        iota16 = lax.iota(jnp.int32, 16)
        zero16 = jnp.zeros((16,), jnp.float32)

        def zrow(i, carry):
            def zcol(j, c2):
                mb[i, pl.ds(j * 16, 16)] = zero16
                return c2
            lax.fori_loop(0, CH // 16, zcol, 0)
            return carry

        lax.fori_loop(0, CHUNK, zrow, 0)

        def zx(i, carry):
            xntab[pl.ds(i * 16, 16)] = zero16
            return carry
        lax.fori_loop(0, XF // 16, zx, 0)

        for t in range(RPT // CHUNK):
            pltpu.sync_copy(mb, hn_sp.at[pl.ds(sid * RPT + t * CHUNK, CHUNK)])

        def load(k, idv, msb):
            off = base + k * CHUNK
            pltpu.async_copy(dst_hbm.at[pl.ds(off, CHUNK)], idv, seml)
            pltpu.async_copy(
                msg_hbm.at[pl.ds(off * XW, CHUNK * XW)], msb, seml)
            pltpu.async_copy(m2_hbm.at[pl.ds(off, CHUNK)], mb, seml)

        def drain_loads(idv, msb):
            pltpu.make_async_copy(
                dst_hbm.at[pl.ds(0, CHUNK)], idv, seml).wait()
            pltpu.make_async_copy(
                msg_hbm.at[pl.ds(0, CHUNK * XW)], msb, seml).wait()
            pltpu.make_async_copy(
                m2_hbm.at[pl.ds(0, CHUNK)], mb, seml).wait()

        def xncompute(idv, msb):
            for g in range(CHUNK // 16):
                rd = idv[pl.ds(g * 16, 16)] * 3
                for j in range(3):
                    vals = msb[pl.ds(j * CHUNK + g * 16, 16)]
                    plsc.addupdate_scatter(xntab, [rd + j], vals)

        load(0, idv0, msb0)
        plsc.subcore_barrier()

        def body(k2, carry):
            ka = k2 * 2
            kb = ka + 1
            drain_loads(idv0, msb0)
            pltpu.sync_copy(mb, hn_sp.at[idv0], add=True)
            load(kb, idv1, msb1)
            xncompute(idv0, msb0)
            drain_loads(idv1, msb1)
            pltpu.sync_copy(mb, hn_sp.at[idv1], add=True)

            @pl.when(k2 < NCHUNK // 2 - 1)
            def _():
                load(ka + 2, idv0, msb0)

            xncompute(idv1, msb1)
            return carry

        lax.fori_loop(0, NCHUNK // 2, body, 0)
        plsc.subcore_barrier()
        pltpu.sync_copy(hn_sp.at[pl.ds(sid * RPT, RPT)],
                        hn_hbm.at[cid, pl.ds(sid * RPT, RPT)])
        pltpu.sync_copy(xntab, xn_hbm.at[wid])

    _sc_cache['gather'] = _sc_gather
    _sc_cache['scatter'] = _sc_scatter
    return _sc_gather, _sc_scatter


def _pre_body(h_ref, at_ref, bt_ref, pk_ref, p_ref, q_ref):
    h = h_ref[...]
    p_ref[...] = _mm(h, at_ref[...])
    q_ref[...] = _mm(h, bt_ref[...]) + pk_ref[0:1, :]


def _edge_body(g_ref, h_ref, xd_ref, w1_ref, w2_ref, pk_ref,
               m2_ref, msg_ref):
    nt = BE // 128
    rad_rows = [xd_ref[4 * t + 3:4 * t + 4, :] for t in range(nt)]
    radial = jnp.concatenate(
        [jnp.transpose(r) for r in rad_rows], axis=0)
    m1 = _silu(g_ref[...] + h_ref[...] + radial * pk_ref[3:4, :])
    m2 = _silu(_mm(m1, w1_ref[...]) + pk_ref[0:1, :])
    m2_ref[...] = m2
    cp = _silu(_mm(m2, w2_ref[...]) + pk_ref[1:2, :])
    scal = jnp.sum(cp * pk_ref[2:3, :], axis=1, keepdims=True)
    for t in range(nt):
        scal_row = jnp.transpose(scal[128 * t:128 * (t + 1), :])
        a = scal_row / (jnp.sqrt(rad_rows[t]) + 1e-30)
        for j in range(3):
            msg_ref[4 * t + j:4 * t + j + 1, :] = (
                xd_ref[4 * t + j:4 * t + j + 1, :] * a)
        msg_ref[4 * t + 3:4 * t + 4, :] = jnp.zeros((1, 128), jnp.float32)


def _node_body(h_ref, hn_ref, x_ref, xn_ref, ct_ref, dt_ref, w1t_ref,
               at_ref, bt_ref, pk_ref,
               hout_ref, xout_ref, p_ref, q_ref):
    hn = hn_ref[0] + hn_ref[1]
    h = h_ref[...]
    u = _silu(_mm(h, ct_ref[...]) + _mm(hn, dt_ref[...]) + pk_ref[0:1, :])
    v = _mm(u, w1t_ref[...]) + pk_ref[1:2, :]
    g = 0.5 * v * (1.0 + lax.erf(v * 0.7071067811865476))
    mu = jnp.mean(g, axis=1, keepdims=True)
    var = jnp.mean((g - mu) ** 2, axis=1, keepdims=True)
    hnew = (g - mu) * lax.rsqrt(var + 1e-5) * pk_ref[2:3, :] + pk_ref[3:4, :]
    hout_ref[...] = hnew
    xout_ref[...] = x_ref[...] + jnp.sum(xn_ref[...], axis=0)
    p_ref[...] = _mm(hnew, at_ref[...])
    q_ref[...] = _mm(hnew, bt_ref[...]) + pk_ref[4:5, :]


def _wspec():
    return pl.BlockSpec((CH, CH), lambda i: (0, 0))


def _pkspec():
    return pl.BlockSpec((8, CH), lambda i: (0, 0))


def _tc_pre(h, at, bt, pk):
    return pl.pallas_call(
        _pre_body,
        grid=(NPAD // BN,),
        in_specs=[
            pl.BlockSpec((BN, CH), lambda i: (i, 0)),
            _wspec(), _wspec(), _pkspec(),
        ],
        out_specs=[
            pl.BlockSpec((BN, CH), lambda i: (i, 0)),
            pl.BlockSpec((BN, CH), lambda i: (i, 0)),
        ],
        out_shape=[
            jax.ShapeDtypeStruct((NPAD, CH), jnp.float32),
            jax.ShapeDtypeStruct((NPAD, CH), jnp.float32),
        ],
    )(h, at, bt, pk)


def _tc_edge(g, h, xd, w1, w2, pk):
    return pl.pallas_call(
        _edge_body,
        grid=(EP // BE,),
        in_specs=[
            pl.BlockSpec((BE, CH), lambda i: (i, 0)),
            pl.BlockSpec((BE, CH), lambda i: (i, 0)),
            pl.BlockSpec((BE * XW // CH, CH), lambda i: (i, 0)),
            _wspec(), _wspec(), _pkspec(),
        ],
        out_specs=[
            pl.BlockSpec((BE, CH), lambda i: (i, 0)),
            pl.BlockSpec((BE * XW // CH, CH), lambda i: (i, 0)),
        ],
        out_shape=[
            jax.ShapeDtypeStruct((EP, CH), jnp.float32),
            jax.ShapeDtypeStruct((EP * XW // CH, CH), jnp.float32),
        ],
    )(g, h, xd, w1, w2, pk)


def _tc_node(h, hn, x, xn, ct, dt, w1t, at, bt, pk):
    return pl.pallas_call(
        _node_body,
        grid=(NPAD // BN,),
        in_specs=[
            pl.BlockSpec((BN, CH), lambda i: (i, 0)),
            pl.BlockSpec((NC, BN, CH), lambda i: (0, i, 0)),
            pl.BlockSpec((BN * 3 // CH, CH), lambda i: (i, 0)),
            pl.BlockSpec((NW, BN * 3 // CH, CH), lambda i: (0, i, 0)),
            _wspec(), _wspec(), _wspec(), _wspec(), _wspec(), _pkspec(),
        ],
        out_specs=[
            pl.BlockSpec((BN, CH), lambda i: (i, 0)),
            pl.BlockSpec((BN * 3 // CH, CH), lambda i: (i, 0)),
            pl.BlockSpec((BN, CH), lambda i: (i, 0)),
            pl.BlockSpec((BN, CH), lambda i: (i, 0)),
        ],
        out_shape=[
            jax.ShapeDtypeStruct((NPAD, CH), jnp.float32),
            jax.ShapeDtypeStruct((XF // CH, CH), jnp.float32),
            jax.ShapeDtypeStruct((NPAD, CH), jnp.float32),
            jax.ShapeDtypeStruct((NPAD, CH), jnp.float32),
        ],
    )(h, hn, x, xn, ct, dt, w1t, at, bt, pk)


def _pack(rows):
    pk = jnp.zeros((8, CH), jnp.float32)
    for r, v in enumerate(rows):
        pk = pk.at[r, :].set(v)
    return pk


def kernel(node_feat, xyz, edge_index, params):
    h = jnp.zeros((NPAD, CH), jnp.float32).at[:N].set(node_feat)
    x = jnp.zeros((NPAD, 3), jnp.float32).at[:N].set(xyz)
    x = x.reshape(XF // CH, CH)
    src = jnp.full((EP,), NPAD - 1, jnp.int32).at[:E].set(edge_index[0])
    dst = jnp.full((EP,), NPAD - 1, jnp.int32).at[:E].set(edge_index[1])

    lp = [params['layer%d' % i] for i in range(2)]
    AT = [p['We0'][:, :CH].T for p in lp]
    BT = [p['We0'][:, CH:2 * CH].T for p in lp]
    WR = [p['We0'][:, 2 * CH] for p in lp]
    pk_edge = [_pack([p['be1'], p['bc0'], p['Wc1'][0], wr])
               for p, wr in zip(lp, WR)]
    W1T = [p['We1'].T for p in lp]
    WC0T = [p['Wc0'].T for p in lp]
    CT = [p['Wn0'][:, :CH].T for p in lp]
    DT = [p['Wn0'][:, CH:].T for p in lp]
    WN1T = [p['Wn1'].T for p in lp]
    WOT = params['Wout'].T

    pk_pre = _pack([lp[0]['be0']])
    pk_node = [
        _pack([lp[0]['bn0'], lp[0]['bn1'], params['ln_g'], params['ln_b'],
               lp[1]['be0']]),
        _pack([lp[1]['bn0'], lp[1]['bn1'], params['ln_g'], params['ln_b'],
               params['bout']]),
    ]
    nxt_at = [AT[1], WOT]
    nxt_bt = [BT[1], WOT]

    _sc_gather, _sc_scatter = _build_sc_kernels()

    P, Q = _tc_pre(h, AT[0], BT[0], pk_pre)
    for i in range(2):
        G, H, XDf = _sc_gather(P, Q, x.reshape(XF), src, dst)
        m2, msg = _tc_edge(G, H, XDf.reshape(EP * XW // CH, CH),
                           W1T[i], WC0T[i], pk_edge[i])
        hn, xn = _sc_scatter(m2, msg.reshape(EP * XW), dst)
        h, x, P, Q = _tc_node(h, hn, x, xn.reshape(NW, XF // CH, CH),
                              CT[i], DT[i], WN1T[i],
                              nxt_at[i], nxt_bt[i], pk_node[i])
    return Q[:N]


# speedup vs baseline: 3.0376x; 1.1826x over previous
import functools

import jax
import jax.numpy as jnp
from jax import lax
from jax.experimental import pallas as pl
from jax.experimental.pallas import tpu as pltpu
from jax.experimental.pallas import tpu_sc as plsc

N = 10000
E = 320000
CH = 128
NPAD = 10240
EP = 327680
XW = 4
NC = 2
NS = 16
NW = NC * NS
EPW = EP // NW
CHUNK = 128
NCHUNK = EPW // CHUNK
RPT = NPAD // NS
XF = NPAD * 3

BE = 2048
BN = 1024


def _silu(x):
    return x * jax.nn.sigmoid(x)


def _mm(a, b):
    return jnp.dot(a, b, preferred_element_type=jnp.float32)


_sc_cache = {}


def _build_sc_kernels(ep):
    if ep in _sc_cache:
        return _sc_cache[ep]
    epw = ep // NW
    nchunk = epw // CHUNK

    mesh = plsc.VectorSubcoreMesh(core_axis_name="c", subcore_axis_name="s")

    @functools.partial(
        pl.kernel,
        mesh=mesh,
        out_type=(
            jax.ShapeDtypeStruct((ep, CH), jnp.float32),
            jax.ShapeDtypeStruct((ep, CH), jnp.float32),
            jax.ShapeDtypeStruct((ep * XW,), jnp.float32),
        ),
        scratch_types=(
            pltpu.VMEM((epw,), jnp.int32),
            pltpu.VMEM((epw,), jnp.int32),
            pltpu.VMEM((CHUNK, CH), jnp.float32),
            pltpu.VMEM((CHUNK, CH), jnp.float32),
            pltpu.VMEM((CHUNK * XW,), jnp.float32),
            pltpu.VMEM((CHUNK, CH), jnp.float32),
            pltpu.VMEM((CHUNK, CH), jnp.float32),
            pltpu.VMEM((CHUNK * XW,), jnp.float32),
            pltpu.VMEM((XF,), jnp.float32),
            pltpu.SemaphoreType.DMA,
            pltpu.SemaphoreType.DMA,
            pltpu.SemaphoreType.DMA,
        ),
        compiler_params=pltpu.CompilerParams(needs_layout_passes=False),
    )
    def _sc_gather(p_hbm, q_hbm, x_hbm, src_hbm, dst_hbm,
                   g_hbm, h_hbm, xd_hbm,
                   s_all, d_all, gb0, hb0, db0, gb1, hb1, db1, xtab,
                   semg, semw0, semw1):
        wid = lax.axis_index("s") * NC + lax.axis_index("c")
        base = wid * epw
        iota16 = lax.iota(jnp.int32, 16)
        pltpu.sync_copy(x_hbm, xtab)
        pltpu.sync_copy(src_hbm.at[pl.ds(base, epw)], s_all)
        pltpu.sync_copy(dst_hbm.at[pl.ds(base, epw)], d_all)

        def issue(k, gb, hb):
            sl = pl.ds(k * CHUNK, CHUNK)
            c1 = pltpu.async_copy(p_hbm.at[s_all.at[sl]], gb, semg)
            c2 = pltpu.async_copy(q_hbm.at[d_all.at[sl]], hb, semg)
            return c1, c2

        def xcompute(k, db):
            for g in range(CHUNK // 16):
                o = k * CHUNK + g * 16
                rs = s_all[pl.ds(o, 16)] * 3
                rd = d_all[pl.ds(o, 16)] * 3
                dx = (plsc.load_gather(xtab, [rs]) -
                      plsc.load_gather(xtab, [rd]))
                dy = (plsc.load_gather(xtab, [rs + 1]) -
                      plsc.load_gather(xtab, [rd + 1]))
                dz_ = (plsc.load_gather(xtab, [rs + 2]) -
                       plsc.load_gather(xtab, [rd + 2]))
                radial = dx * dx + dy * dy + dz_ * dz_
                db[pl.ds(g * 16, 16)] = dx
                db[pl.ds(CHUNK + g * 16, 16)] = dy
                db[pl.ds(2 * CHUNK + g * 16, 16)] = dz_
                db[pl.ds(3 * CHUNK + g * 16, 16)] = radial

        def write(k, gb, hb, db, semw):
            off = base + k * CHUNK
            pltpu.async_copy(gb, g_hbm.at[pl.ds(off, CHUNK)], semw)
            pltpu.async_copy(hb, h_hbm.at[pl.ds(off, CHUNK)], semw)
            pltpu.async_copy(db, xd_hbm.at[pl.ds(off * XW, CHUNK * XW)], semw)

        def drain(gb, hb, db, semw):
            pltpu.make_async_copy(g_hbm.at[pl.ds(0, CHUNK)], gb, semw).wait()
            pltpu.make_async_copy(h_hbm.at[pl.ds(0, CHUNK)], hb, semw).wait()
            pltpu.make_async_copy(
                xd_hbm.at[pl.ds(0, CHUNK * XW)], db, semw).wait()

        def body(k2, carry):
            ka = k2 * 2
            kb = ka + 1

            @pl.when(k2 > 0)
            def _():
                drain(gb0, hb0, db0, semw0)

            ca1, ca2 = issue(ka, gb0, hb0)

            @pl.when(k2 > 0)
            def _():
                drain(gb1, hb1, db1, semw1)

            cb1, cb2 = issue(kb, gb1, hb1)
            xcompute(ka, db0)
            xcompute(kb, db1)
            ca1.wait()
            ca2.wait()
            write(ka, gb0, hb0, db0, semw0)
            cb1.wait()
            cb2.wait()
            write(kb, gb1, hb1, db1, semw1)
            return carry

        lax.fori_loop(0, nchunk // 2, body, 0)
        drain(gb0, hb0, db0, semw0)
        drain(gb1, hb1, db1, semw1)

    @functools.partial(
        pl.kernel,
        mesh=mesh,
        out_type=(
            jax.ShapeDtypeStruct((NC, NPAD, CH), jnp.float32),
            jax.ShapeDtypeStruct((NW, XF), jnp.float32),
        ),
        scratch_types=(
            pltpu.VMEM((CHUNK,), jnp.int32),
            pltpu.VMEM((CHUNK,), jnp.int32),
            pltpu.VMEM((CHUNK, CH), jnp.float32),
            pltpu.VMEM((CHUNK * XW,), jnp.float32),
            pltpu.VMEM((CHUNK * XW,), jnp.float32),
            pltpu.VMEM((XF,), jnp.float32),
            pltpu.VMEM_SHARED((NPAD, CH), jnp.float32),
            pltpu.SemaphoreType.DMA,
        ),
        compiler_params=pltpu.CompilerParams(needs_layout_passes=False),
    )
    def _sc_scatter(m2_hbm, msg_hbm, dst_hbm,
                    hn_hbm, xn_hbm,
                    idv0, idv1, mb, msb0, msb1, xntab, hn_sp, seml):
        cid = lax.axis_index("c")
        sid = lax.axis_index("s")
        wid = sid * NC + cid
        base = wid * epw
        iota16 = lax.iota(jnp.int32, 16)
        zero16 = jnp.zeros((16,), jnp.float32)

        def zrow(i, carry):
            def zcol(j, c2):
                mb[i, pl.ds(j * 16, 16)] = zero16
                return c2
            lax.fori_loop(0, CH // 16, zcol, 0)
            return carry

        lax.fori_loop(0, CHUNK, zrow, 0)

        def zx(i, carry):
            xntab[pl.ds(i * 16, 16)] = zero16
            return carry
        lax.fori_loop(0, XF // 16, zx, 0)

        for t in range(RPT // CHUNK):
            pltpu.sync_copy(mb, hn_sp.at[pl.ds(sid * RPT + t * CHUNK, CHUNK)])

        def load(k, idv, msb):
            off = base + k * CHUNK
            pltpu.async_copy(dst_hbm.at[pl.ds(off, CHUNK)], idv, seml)
            pltpu.async_copy(
                msg_hbm.at[pl.ds(off * XW, CHUNK * XW)], msb, seml)
            pltpu.async_copy(m2_hbm.at[pl.ds(off, CHUNK)], mb, seml)

        def drain_loads(idv, msb):
            pltpu.make_async_copy(
                dst_hbm.at[pl.ds(0, CHUNK)], idv, seml).wait()
            pltpu.make_async_copy(
                msg_hbm.at[pl.ds(0, CHUNK * XW)], msb, seml).wait()
            pltpu.make_async_copy(
                m2_hbm.at[pl.ds(0, CHUNK)], mb, seml).wait()

        def xncompute(idv, msb):
            for g in range(CHUNK // 16):
                rd = idv[pl.ds(g * 16, 16)] * 3
                for j in range(3):
                    vals = msb[pl.ds(j * CHUNK + g * 16, 16)]
                    plsc.addupdate_scatter(xntab, [rd + j], vals)

        load(0, idv0, msb0)
        plsc.subcore_barrier()

        def body(k2, carry):
            ka = k2 * 2
            kb = ka + 1
            drain_loads(idv0, msb0)
            pltpu.sync_copy(mb, hn_sp.at[idv0], add=True)
            load(kb, idv1, msb1)
            xncompute(idv0, msb0)
            drain_loads(idv1, msb1)
            pltpu.sync_copy(mb, hn_sp.at[idv1], add=True)

            @pl.when(k2 < nchunk // 2 - 1)
            def _():
                load(ka + 2, idv0, msb0)

            xncompute(idv1, msb1)
            return carry

        lax.fori_loop(0, nchunk // 2, body, 0)
        plsc.subcore_barrier()
        pltpu.sync_copy(hn_sp.at[pl.ds(sid * RPT, RPT)],
                        hn_hbm.at[cid, pl.ds(sid * RPT, RPT)])
        pltpu.sync_copy(xntab, xn_hbm.at[wid])

    _sc_cache[ep] = (_sc_gather, _sc_scatter)
    return _sc_cache[ep]


def _pre_body(h_ref, at_ref, bt_ref, pk_ref, p_ref, q_ref):
    h = h_ref[...]
    p_ref[...] = _mm(h, at_ref[...])
    q_ref[...] = _mm(h, bt_ref[...]) + pk_ref[0:1, :]


def _edge_body(g_ref, h_ref, xd_ref, w1_ref, w2_ref, pk_ref,
               m2_ref, msg_ref):
    nt = BE // 128
    rad_rows = [xd_ref[4 * t + 3:4 * t + 4, :] for t in range(nt)]
    radial = jnp.concatenate(
        [jnp.transpose(r) for r in rad_rows], axis=0)
    m1 = _silu(g_ref[...] + h_ref[...] + radial * pk_ref[3:4, :])
    m2 = _silu(_mm(m1, w1_ref[...]) + pk_ref[0:1, :])
    m2_ref[...] = m2
    cp = _silu(_mm(m2, w2_ref[...]) + pk_ref[1:2, :])
    scal = jnp.sum(cp * pk_ref[2:3, :], axis=1, keepdims=True)
    for t in range(nt):
        scal_row = jnp.transpose(scal[128 * t:128 * (t + 1), :])
        a = scal_row / (jnp.sqrt(rad_rows[t]) + 1e-30)
        for j in range(3):
            msg_ref[4 * t + j:4 * t + j + 1, :] = (
                xd_ref[4 * t + j:4 * t + j + 1, :] * a)
        msg_ref[4 * t + 3:4 * t + 4, :] = jnp.zeros((1, 128), jnp.float32)


def _node_body(h_ref, hn_ref, hn1_ref, x_ref, xn_ref, xn1_ref,
               ct_ref, dt_ref, w1t_ref, at_ref, bt_ref, pk_ref,
               hout_ref, xout_ref, p_ref, q_ref):
    hn = hn_ref[0] + hn_ref[1] + hn1_ref[0] + hn1_ref[1]
    h = h_ref[...]
    u = _silu(_mm(h, ct_ref[...]) + _mm(hn, dt_ref[...]) + pk_ref[0:1, :])
    v = _mm(u, w1t_ref[...]) + pk_ref[1:2, :]
    g = 0.5 * v * (1.0 + lax.erf(v * 0.7071067811865476))
    mu = jnp.mean(g, axis=1, keepdims=True)
    var = jnp.mean((g - mu) ** 2, axis=1, keepdims=True)
    hnew = (g - mu) * lax.rsqrt(var + 1e-5) * pk_ref[2:3, :] + pk_ref[3:4, :]
    hout_ref[...] = hnew
    xout_ref[...] = (x_ref[...] + jnp.sum(xn_ref[...], axis=0)
                     + jnp.sum(xn1_ref[...], axis=0))
    p_ref[...] = _mm(hnew, at_ref[...])
    q_ref[...] = _mm(hnew, bt_ref[...]) + pk_ref[4:5, :]


def _wspec():
    return pl.BlockSpec((CH, CH), lambda i: (0, 0))


def _pkspec():
    return pl.BlockSpec((8, CH), lambda i: (0, 0))


def _tc_pre(h, at, bt, pk):
    return pl.pallas_call(
        _pre_body,
        grid=(NPAD // BN,),
        in_specs=[
            pl.BlockSpec((BN, CH), lambda i: (i, 0)),
            _wspec(), _wspec(), _pkspec(),
        ],
        out_specs=[
            pl.BlockSpec((BN, CH), lambda i: (i, 0)),
            pl.BlockSpec((BN, CH), lambda i: (i, 0)),
        ],
        out_shape=[
            jax.ShapeDtypeStruct((NPAD, CH), jnp.float32),
            jax.ShapeDtypeStruct((NPAD, CH), jnp.float32),
        ],
    )(h, at, bt, pk)


def _tc_edge(g, h, xd, w1, w2, pk):
    ep = g.shape[0]
    return pl.pallas_call(
        _edge_body,
        grid=(ep // BE,),
        in_specs=[
            pl.BlockSpec((BE, CH), lambda i: (i, 0)),
            pl.BlockSpec((BE, CH), lambda i: (i, 0)),
            pl.BlockSpec((BE * XW // CH, CH), lambda i: (i, 0)),
            _wspec(), _wspec(), _pkspec(),
        ],
        out_specs=[
            pl.BlockSpec((BE, CH), lambda i: (i, 0)),
            pl.BlockSpec((BE * XW // CH, CH), lambda i: (i, 0)),
        ],
        out_shape=[
            jax.ShapeDtypeStruct((ep, CH), jnp.float32),
            jax.ShapeDtypeStruct((ep * XW // CH, CH), jnp.float32),
        ],
    )(g, h, xd, w1, w2, pk)


def _tc_node(h, hn, hn1, x, xn, xn1, ct, dt, w1t, at, bt, pk):
    return pl.pallas_call(
        _node_body,
        grid=(NPAD // BN,),
        in_specs=[
            pl.BlockSpec((BN, CH), lambda i: (i, 0)),
            pl.BlockSpec((NC, BN, CH), lambda i: (0, i, 0)),
            pl.BlockSpec((NC, BN, CH), lambda i: (0, i, 0)),
            pl.BlockSpec((BN * 3 // CH, CH), lambda i: (i, 0)),
            pl.BlockSpec((NW, BN * 3 // CH, CH), lambda i: (0, i, 0)),
            pl.BlockSpec((NW, BN * 3 // CH, CH), lambda i: (0, i, 0)),
            _wspec(), _wspec(), _wspec(), _wspec(), _wspec(), _pkspec(),
        ],
        out_specs=[
            pl.BlockSpec((BN, CH), lambda i: (i, 0)),
            pl.BlockSpec((BN * 3 // CH, CH), lambda i: (i, 0)),
            pl.BlockSpec((BN, CH), lambda i: (i, 0)),
            pl.BlockSpec((BN, CH), lambda i: (i, 0)),
        ],
        out_shape=[
            jax.ShapeDtypeStruct((NPAD, CH), jnp.float32),
            jax.ShapeDtypeStruct((XF // CH, CH), jnp.float32),
            jax.ShapeDtypeStruct((NPAD, CH), jnp.float32),
            jax.ShapeDtypeStruct((NPAD, CH), jnp.float32),
        ],
    )(h, hn, hn1, x, xn, xn1, ct, dt, w1t, at, bt, pk)


def _pack(rows):
    pk = jnp.zeros((8, CH), jnp.float32)
    for r, v in enumerate(rows):
        pk = pk.at[r, :].set(v)
    return pk


def kernel(node_feat, xyz, edge_index, params):
    h = jnp.zeros((NPAD, CH), jnp.float32).at[:N].set(node_feat)
    x = jnp.zeros((NPAD, 3), jnp.float32).at[:N].set(xyz)
    x = x.reshape(XF // CH, CH)
    src = jnp.full((EP,), NPAD - 1, jnp.int32).at[:E].set(edge_index[0])
    dst = jnp.full((EP,), NPAD - 1, jnp.int32).at[:E].set(edge_index[1])

    lp = [params['layer%d' % i] for i in range(2)]
    AT = [p['We0'][:, :CH].T for p in lp]
    BT = [p['We0'][:, CH:2 * CH].T for p in lp]
    WR = [p['We0'][:, 2 * CH] for p in lp]
    pk_edge = [_pack([p['be1'], p['bc0'], p['Wc1'][0], wr])
               for p, wr in zip(lp, WR)]
    W1T = [p['We1'].T for p in lp]
    WC0T = [p['Wc0'].T for p in lp]
    CT = [p['Wn0'][:, :CH].T for p in lp]
    DT = [p['Wn0'][:, CH:].T for p in lp]
    WN1T = [p['Wn1'].T for p in lp]
    WOT = params['Wout'].T

    pk_pre = _pack([lp[0]['be0']])
    pk_node = [
        _pack([lp[0]['bn0'], lp[0]['bn1'], params['ln_g'], params['ln_b'],
               lp[1]['be0']]),
        _pack([lp[1]['bn0'], lp[1]['bn1'], params['ln_g'], params['ln_b'],
               params['bout']]),
    ]
    nxt_at = [AT[1], WOT]
    nxt_bt = [BT[1], WOT]

    eh = EP // 2
    _sc_gather, _sc_scatter = _build_sc_kernels(eh)
    srcs = [src[:eh], src[eh:]]
    dsts = [dst[:eh], dst[eh:]]

    P, Q = _tc_pre(h, AT[0], BT[0], pk_pre)
    for i in range(2):
        xf = x.reshape(XF)
        hns, xns = [], []
        for s in range(2):
            G, H, XDf = _sc_gather(P, Q, xf, srcs[s], dsts[s])
            m2, msg = _tc_edge(G, H, XDf.reshape(eh * XW // CH, CH),
                               W1T[i], WC0T[i], pk_edge[i])
            hn_s, xn_s = _sc_scatter(m2, msg.reshape(eh * XW), dsts[s])
            hns.append(hn_s)
            xns.append(xn_s.reshape(NW, XF // CH, CH))
        h, x, P, Q = _tc_node(h, hns[0], hns[1], x, xns[0], xns[1],
                              CT[i], DT[i], WN1T[i],
                              nxt_at[i], nxt_bt[i], pk_node[i])
    return Q[:N]
